# baseline (speedup 1.0000x reference)
import functools

import jax
import jax.numpy as jnp
from jax import lax
from jax.experimental import pallas as pl
from jax.experimental.pallas import tpu as pltpu
from jax.experimental.pallas import tpu_sc as plsc

_EMB = 128
_BATCH = 16384
_VOCAB = 3

_INFO = plsc.get_sparse_core_info()
_NC = _INFO.num_cores
_NS = _INFO.num_subcores
_NW = _NC * _NS
_BPW = _BATCH // _NW
_L = _INFO.num_lanes
_NCHW = _EMB // _L
_GSZ = _L * _EMB
_NGRP = _BPW // _L

_mesh = plsc.VectorSubcoreMesh(core_axis_name="c", subcore_axis_name="s")


@functools.partial(
    pl.kernel,
    mesh=_mesh,
    compiler_params=pltpu.CompilerParams(needs_layout_passes=False),
    out_type=jax.ShapeDtypeStruct((_BATCH * _EMB,), jnp.float32),
    scratch_types=[
        pltpu.VMEM((_BPW,), jnp.int32),
        pltpu.VMEM((_VOCAB * _EMB,), jnp.float32),
        pltpu.VMEM((_BPW * _EMB,), jnp.float32),
        pltpu.SemaphoreType.DMA,
        pltpu.SemaphoreType.DMA,
    ],
)
def _embed_lookup(idx_hbm, table_hbm, out_hbm, idx_v, table_v, out_v, sem_in, sem_out):
    wid = lax.axis_index("s") * _NC + lax.axis_index("c")
    cp_tab = pltpu.async_copy(table_hbm, table_v, sem_in)
    pltpu.sync_copy(idx_hbm.at[wid], idx_v)
    cp_tab.wait()
    out_base = wid * (_BPW * _EMB)

    @plsc.parallel_loop(0, _NGRP, unroll=1)
    def _group(g):
        gbase = g * _GSZ
        vidx = idx_v[pl.ds(g * _L, _L)]
        for j in range(_L):
            off = vidx[j] * _EMB
            base = gbase + j * _EMB
            for c in range(_NCHW):
                out_v[pl.ds(base + c * _L, _L)] = table_v[pl.ds(off + c * _L, _L)]
        pltpu.async_copy(
            out_v.at[pl.ds(gbase, _GSZ)],
            out_hbm.at[pl.ds(out_base + gbase, _GSZ)],
            sem_out,
        )

    pltpu.make_async_copy(
        out_hbm.at[pl.ds(out_base, _BPW * _EMB)], out_v, sem_out
    ).wait()


def kernel(x, table):
    idx = x.reshape(_NW, _BPW).astype(jnp.int32)
    flat = _embed_lookup(idx, table.reshape(_VOCAB * _EMB))
    return flat.reshape(_BATCH, _EMB)
